# baseline (speedup 1.0000x reference)
import functools

import jax
import jax.numpy as jnp
from jax import lax
from jax.experimental import pallas as pl
from jax.experimental.pallas import tpu as pltpu
from jax.experimental.pallas import tpu_sc as plsc

B = 16384
D = 32
NC = 2
NS = 16
NW = NC * NS
BPW = B // NW
L = 16


def _bpr_body(user_hbm, i_hbm, j_hbm, ut_hbm, it_hbm, out_hbm,
              idx_u, idx_i, idx_j, u_rows, i_rows, j_rows, out_v, sem):
    wid = lax.axis_index("s") * NC + lax.axis_index("c")
    base = wid * BPW

    pltpu.sync_copy(user_hbm.at[pl.ds(base, BPW)], idx_u)
    pltpu.sync_copy(i_hbm.at[pl.ds(base, BPW)], idx_i)
    pltpu.sync_copy(j_hbm.at[pl.ds(base, BPW)], idx_j)

    cu = pltpu.async_copy(ut_hbm.at[idx_u], u_rows, sem)
    ci = pltpu.async_copy(it_hbm.at[idx_i], i_rows, sem)
    cj = pltpu.async_copy(it_hbm.at[idx_j], j_rows, sem)
    cu.wait()
    ci.wait()
    cj.wait()

    def block(blk, _):
        rows = blk * L + lax.iota(jnp.int32, L)
        acc = jnp.zeros((L,), jnp.float32)
        for dd in range(D):
            dvec = jnp.full((L,), dd, jnp.int32)
            u_v = plsc.load_gather(u_rows, [rows, dvec])
            i_v = plsc.load_gather(i_rows, [rows, dvec])
            j_v = plsc.load_gather(j_rows, [rows, dvec])
            acc = acc + u_v * (i_v - j_v)
        out_v[pl.ds(blk * L, L)] = acc
        return 0

    lax.fori_loop(0, BPW // L, block, 0)

    pltpu.sync_copy(out_v, out_hbm.at[pl.ds(base, BPW)])


_bpr_kernel = functools.partial(
    pl.kernel,
    out_type=jax.ShapeDtypeStruct((B,), jnp.float32),
    mesh=plsc.VectorSubcoreMesh(core_axis_name="c", subcore_axis_name="s"),
    scratch_types=[
        pltpu.VMEM((BPW,), jnp.int32),
        pltpu.VMEM((BPW,), jnp.int32),
        pltpu.VMEM((BPW,), jnp.int32),
        pltpu.VMEM((BPW, D), jnp.float32),
        pltpu.VMEM((BPW, D), jnp.float32),
        pltpu.VMEM((BPW, D), jnp.float32),
        pltpu.VMEM((BPW,), jnp.float32),
        pltpu.SemaphoreType.DMA,
    ],
    compiler_params=pltpu.CompilerParams(
        needs_layout_passes=False, use_tc_tiling_on_sc=False),
)(_bpr_body)


def kernel(user, i, j, user_table, item_table):
    return _bpr_kernel(user, i, j, user_table, item_table)
